# baseline (speedup 1.0000x reference)
import functools

import jax
import jax.numpy as jnp
from jax import lax
from jax.experimental import pallas as pl
from jax.experimental.pallas import tpu as pltpu, tpu_sc as plsc

N = 4096
NP = 1_000_000
MAXPD = 20
NW = 32
NB = 256
ROWS_PB = N // NB
CHUNK = ROWS_PB * N
VREGS = NP // 16
QV = -(-VREGS // NW)
WIN = 64
NFULL = QV // WIN
TAIL = QV - NFULL * WIN
IDXB = 1024
NPAD = NP + 8 * NB + WIN * 16 + 16
W3 = 2048

_mesh = plsc.VectorSubcoreMesh(core_axis_name="c", subcore_axis_name="s")
_params = pltpu.CompilerParams(needs_layout_passes=False)


def _wid():
    return lax.axis_index("c") * 16 + lax.axis_index("s")


def _splat(x):
    return jnp.full((16,), x, jnp.int32)


def _vext0(vec):
    lane = lax.iota(jnp.int32, 16)
    return jnp.sum(jnp.where(lane == 0, vec, jnp.zeros((16,), vec.dtype)))


def _sget(ref, idx_scalar):
    return _vext0(plsc.load_gather(ref, [_splat(idx_scalar)]))


@functools.partial(
    pl.kernel,
    out_type=jax.ShapeDtypeStruct((NW * NB,), jnp.int32),
    mesh=_mesh,
    compiler_params=_params,
    scratch_types=[
        pltpu.VMEM((WIN * 16,), jnp.int32),
        pltpu.VMEM((NB,), jnp.int32),
    ],
)
def _k1(src_hbm, hist_out, win_ref, hist_ref):
    w = _wid()
    base = w * QV
    ones = jnp.ones((16,), jnp.int32)
    for i in range(NB // 16):
        hist_ref[pl.ds(i * 16, 16)] = jnp.zeros((16,), jnp.int32)

    def fullwin(j, carry):
        pltpu.sync_copy(src_hbm.at[pl.ds((base + j * WIN) * 16, WIN * 16)],
                        win_ref)
        for v in range(WIN):
            srcv = win_ref[pl.ds(v * 16, 16)]
            plsc.addupdate_scatter(hist_ref, [srcv >> 4], ones)
        return carry

    lax.fori_loop(0, NFULL, fullwin, 0)

    tb = jnp.minimum(base + NFULL * WIN, VREGS - TAIL)
    vlo = base + NFULL * WIN - tb
    pltpu.sync_copy(src_hbm.at[pl.ds(tb * 16, TAIL * 16)],
                    win_ref.at[pl.ds(0, TAIL * 16)])
    for v in range(TAIL):
        m = _splat(v) >= _splat(vlo)
        srcv = win_ref[pl.ds(v * 16, 16)]
        plsc.addupdate_scatter(hist_ref, [srcv >> 4], ones, mask=m)

    pltpu.sync_copy(hist_ref, hist_out.at[pl.ds(w * NB, NB)])


@functools.partial(
    pl.kernel,
    out_type=jax.ShapeDtypeStruct((NPAD,), jnp.int32),
    mesh=_mesh,
    compiler_params=_params,
    scratch_types=[
        pltpu.VMEM((NW * NB,), jnp.int32),
        pltpu.VMEM((NB,), jnp.int32),
        pltpu.VMEM((WIN * 16,), jnp.int32),
        pltpu.VMEM((WIN * 16,), jnp.int32),
        pltpu.VMEM((WIN * 16,), jnp.int32),
        pltpu.VMEM((2, WIN * 16 // IDXB, IDXB), jnp.int32),
        pltpu.VMEM((2, WIN * 16 // IDXB, IDXB), jnp.int32),
        pltpu.SemaphoreType.DMA,
        pltpu.SemaphoreType.DMA,
    ],
)
def _k2(src_hbm, dst_hbm, len_hbm, hist_hbm, pair_out,
        hist_ref, cur_ref, swin, dwin, lwin, pstage, ostage, sem, osem):
    w = _wid()
    wv = _splat(w)
    pltpu.sync_copy(hist_hbm, hist_ref)

    carry = jnp.int32(0)
    for bv in range(NB // 16):
        tot = jnp.zeros((16,), jnp.int32)
        pre = jnp.zeros((16,), jnp.int32)
        for wi in range(NW):
            h = hist_ref[pl.ds(wi * NB + bv * 16, 16)]
            tot = tot + h
            pre = pre + jnp.where(_splat(wi) < wv, h, jnp.zeros((16,), jnp.int32))
        tota = jnp.bitwise_and(tot + 7, _splat(-8))
        cs = plsc.cumsum(tota)
        cur_ref[pl.ds(bv * 16, 16)] = cs - tota + _splat(carry) + pre
        carry = carry + jnp.sum(tota)

    base = w * QV
    nrows = WIN * 16 // IDXB

    def _emit_vreg(srcv, dstv, lenv, mask):
        bucket = srcv >> 4
        cellv = ((srcv & 15) << 12) | dstv
        bidx = jnp.minimum(jnp.maximum(lenv, 1), MAXPD) - 1
        packed = (cellv << 5) | bidx
        if mask is None:
            cnt, _ = plsc.scan_count(bucket)
            cur = plsc.load_gather(cur_ref, [bucket])
            dest = cur + cnt - 1
            plsc.store_scatter(cur_ref, [bucket], cur + cnt)
        else:
            cnt, _ = plsc.scan_count(bucket, mask=mask)
            cur = plsc.load_gather(cur_ref, [bucket])
            dest = cur + cnt - 1
            plsc.store_scatter(cur_ref, [bucket], cur + cnt, mask=mask)
            pad = _splat(NPAD - 16) + lax.iota(jnp.int32, 16)
            dest = jnp.where(mask, dest, pad)
        return packed, dest

    def _do_window(j, slot):
        off = (base + j * WIN) * 16
        h1 = pltpu.async_copy(src_hbm.at[pl.ds(off, WIN * 16)], swin, sem)
        h2 = pltpu.async_copy(dst_hbm.at[pl.ds(off, WIN * 16)], dwin, sem)
        h3 = pltpu.async_copy(len_hbm.at[pl.ds(off, WIN * 16)], lwin, sem)
        h1.wait(); h2.wait(); h3.wait()
        for v in range(WIN):
            packed, dest = _emit_vreg(
                swin[pl.ds(v * 16, 16)], dwin[pl.ds(v * 16, 16)],
                lwin[pl.ds(v * 16, 16)], None)
            r, q = (v * 16) // IDXB, (v * 16) % IDXB
            pstage[slot, r, pl.ds(q, 16)] = packed
            ostage[slot, r, pl.ds(q, 16)] = dest
        for r in range(nrows):
            pltpu.async_copy(pstage.at[slot, r],
                             pair_out.at[ostage.at[slot, r]], osem)

    def _drain(slot):
        for r in range(nrows):
            pltpu.make_async_copy(pstage.at[slot, r],
                                  pair_out.at[ostage.at[slot, r]], osem).wait()

    def pairwin(i, carry):
        @pl.when(i > 0)
        def _():
            _drain(0)
        _do_window(2 * i, 0)

        @pl.when(i > 0)
        def _():
            _drain(1)
        _do_window(2 * i + 1, 1)
        return carry

    lax.fori_loop(0, NFULL // 2, pairwin, 0)
    _drain(0)
    _drain(1)

    tb = jnp.minimum(base + NFULL * WIN, VREGS - TAIL)
    vlo = base + NFULL * WIN - tb
    pltpu.sync_copy(src_hbm.at[pl.ds(tb * 16, TAIL * 16)],
                    swin.at[pl.ds(0, TAIL * 16)])
    pltpu.sync_copy(dst_hbm.at[pl.ds(tb * 16, TAIL * 16)],
                    dwin.at[pl.ds(0, TAIL * 16)])
    pltpu.sync_copy(len_hbm.at[pl.ds(tb * 16, TAIL * 16)],
                    lwin.at[pl.ds(0, TAIL * 16)])
    for v in range(TAIL):
        m = _splat(v) >= _splat(vlo)
        packed, dest = _emit_vreg(
            swin[pl.ds(v * 16, 16)], dwin[pl.ds(v * 16, 16)],
            lwin[pl.ds(v * 16, 16)], m)
        r, q = (v * 16) // IDXB, (v * 16) % IDXB
        pstage[0, r, pl.ds(q, 16)] = packed
        ostage[0, r, pl.ds(q, 16)] = dest
    for v in range(TAIL, ((TAIL * 16 + IDXB - 1) // IDXB) * IDXB // 16):
        r, q = (v * 16) // IDXB, (v * 16) % IDXB
        ostage[0, r, pl.ds(q, 16)] = _splat(NPAD - 16) + lax.iota(jnp.int32, 16)
    handles = []
    for r in range(-(-TAIL * 16 // IDXB)):
        handles.append(pltpu.async_copy(
            pstage.at[0, r], pair_out.at[ostage.at[0, r]], osem))
    for h in handles:
        h.wait()


@functools.partial(
    pl.kernel,
    out_type=jax.ShapeDtypeStruct((N * N,), jnp.float32),
    mesh=_mesh,
    compiler_params=_params,
    scratch_types=[
        pltpu.VMEM((CHUNK,), jnp.float32),
        pltpu.VMEM((NW * NB,), jnp.int32),
        pltpu.VMEM((NB,), jnp.int32),
        pltpu.VMEM((NB,), jnp.int32),
        pltpu.VMEM((W3,), jnp.int32),
        pltpu.VMEM((32,), jnp.float32),
        pltpu.SemaphoreType.DMA,
    ],
)
def _k3(pair_hbm, hist_hbm, b_hbm, out_hbm,
        chunk, hist_ref, off_ref, tot_ref, pwin, b_ref, sem):
    w = _wid()
    pltpu.sync_copy(hist_hbm, hist_ref)
    pltpu.sync_copy(b_hbm, b_ref.at[pl.ds(0, MAXPD)])

    carry = jnp.int32(0)
    for bv in range(NB // 16):
        tot = jnp.zeros((16,), jnp.int32)
        for wi in range(NW):
            tot = tot + hist_ref[pl.ds(wi * NB + bv * 16, 16)]
        tota = jnp.bitwise_and(tot + 7, _splat(-8))
        cs = plsc.cumsum(tota)
        off_ref[pl.ds(bv * 16, 16)] = cs - tota + _splat(carry)
        tot_ref[pl.ds(bv * 16, 16)] = tot
        carry = carry + jnp.sum(tota)

    def zbody(i, carry):
        for k in range(16):
            chunk[pl.ds(i * 256 + k * 16, 16)] = jnp.zeros((16,), jnp.float32)
        return carry

    lax.fori_loop(0, CHUNK // 256, zbody, 0)

    lane = lax.iota(jnp.int32, 16)
    zerof = jnp.zeros((16,), jnp.float32)

    def round_body(r, rcarry):
        b = w * (NB // NW) + r
        start = pl.multiple_of(_sget(off_ref, b), 8)
        tot = _sget(tot_ref, b)
        nwin = (tot + W3 - 1) // W3

        def swin_body(j, carry):
            rem = tot - j * W3
            pltpu.sync_copy(pair_hbm.at[pl.ds(start + j * W3, W3)], pwin)
            for v in range(W3 // 16):
                m = (_splat(v * 16) + lane) < _splat(rem)
                p = pwin[pl.ds(v * 16, 16)]
                val = plsc.load_gather(b_ref, [p & 31])
                plsc.store_scatter(chunk, [p >> 5], val, mask=m)
            return carry

        lax.fori_loop(0, nwin, swin_body, 0)

        pltpu.sync_copy(chunk,
                        out_hbm.at[pl.ds(pl.multiple_of(b * CHUNK, 8), CHUNK)])

        def zwin_body(j, carry):
            rem = tot - j * W3
            pltpu.sync_copy(pair_hbm.at[pl.ds(start + j * W3, W3)], pwin)
            for v in range(W3 // 16):
                m = (_splat(v * 16) + lane) < _splat(rem)
                plsc.store_scatter(chunk, [pwin[pl.ds(v * 16, 16)] >> 5],
                                   zerof, mask=m)
            return carry

        lax.fori_loop(0, nwin, zwin_body, 0)
        return rcarry

    lax.fori_loop(0, NB // NW, round_body, 0)


def kernel(x, path_src, path_dst, path_len, b):
    hist = _k1(path_src)
    pair = _k2(path_src, path_dst, path_len, hist)
    flat = _k3(pair, hist, b)
    return flat.reshape(N, N).astype(x.dtype)


# speedup vs baseline: 2.1508x; 2.1508x over previous
import functools

import jax
import jax.numpy as jnp
from jax import lax
from jax.experimental import pallas as pl
from jax.experimental.pallas import tpu as pltpu, tpu_sc as plsc

N = 4096
NP = 1_000_000
MAXPD = 20
NW = 32
NB = 256
ROWS_PB = N // NB
CHUNK = ROWS_PB * N
VREGS = NP // 16
QV = -(-VREGS // NW)
WIN = 64
NFULL = QV // WIN
TAIL = QV - NFULL * WIN
IDXB = 128
NPAD = NP + 8 * NB + WIN * 16 + 16
W3 = 2048

_mesh = plsc.VectorSubcoreMesh(core_axis_name="c", subcore_axis_name="s")
_params = pltpu.CompilerParams(needs_layout_passes=False)


def _wid():
    return lax.axis_index("c") * 16 + lax.axis_index("s")


def _splat(x):
    return jnp.full((16,), x, jnp.int32)


def _vext0(vec):
    lane = lax.iota(jnp.int32, 16)
    return jnp.sum(jnp.where(lane == 0, vec, jnp.zeros((16,), vec.dtype)))


def _sget(ref, idx_scalar):
    return _vext0(plsc.load_gather(ref, [_splat(idx_scalar)]))


@functools.partial(
    pl.kernel,
    out_type=jax.ShapeDtypeStruct((NW * NB,), jnp.int32),
    mesh=_mesh,
    compiler_params=_params,
    scratch_types=[
        pltpu.VMEM((WIN * 16,), jnp.int32),
        pltpu.VMEM((NB,), jnp.int32),
    ],
)
def _k1(src_hbm, hist_out, win_ref, hist_ref):
    w = _wid()
    base = w * QV
    ones = jnp.ones((16,), jnp.int32)
    for i in range(NB // 16):
        hist_ref[pl.ds(i * 16, 16)] = jnp.zeros((16,), jnp.int32)

    def fullwin(j, carry):
        pltpu.sync_copy(src_hbm.at[pl.ds((base + j * WIN) * 16, WIN * 16)],
                        win_ref)
        for v in range(WIN):
            srcv = win_ref[pl.ds(v * 16, 16)]
            plsc.addupdate_scatter(hist_ref, [srcv >> 4], ones)
        return carry

    lax.fori_loop(0, NFULL, fullwin, 0)

    tb = jnp.minimum(base + NFULL * WIN, VREGS - TAIL)
    vlo = base + NFULL * WIN - tb
    pltpu.sync_copy(src_hbm.at[pl.ds(tb * 16, TAIL * 16)],
                    win_ref.at[pl.ds(0, TAIL * 16)])
    for v in range(TAIL):
        m = _splat(v) >= _splat(vlo)
        srcv = win_ref[pl.ds(v * 16, 16)]
        plsc.addupdate_scatter(hist_ref, [srcv >> 4], ones, mask=m)

    pltpu.sync_copy(hist_ref, hist_out.at[pl.ds(w * NB, NB)])


@functools.partial(
    pl.kernel,
    out_type=jax.ShapeDtypeStruct((NPAD,), jnp.int32),
    mesh=_mesh,
    compiler_params=_params,
    scratch_types=[
        pltpu.VMEM((NW * NB,), jnp.int32),
        pltpu.VMEM((NB,), jnp.int32),
        pltpu.VMEM((WIN * 16,), jnp.int32),
        pltpu.VMEM((WIN * 16,), jnp.int32),
        pltpu.VMEM((WIN * 16,), jnp.int32),
        pltpu.VMEM((2, WIN * 16 // IDXB, IDXB), jnp.int32),
        pltpu.VMEM((2, WIN * 16 // IDXB, IDXB), jnp.int32),
        pltpu.SemaphoreType.DMA,
        pltpu.SemaphoreType.DMA,
    ],
)
def _k2(src_hbm, dst_hbm, len_hbm, hist_hbm, pair_out,
        hist_ref, cur_ref, swin, dwin, lwin, pstage, ostage, sem, osem):
    w = _wid()
    wv = _splat(w)
    pltpu.sync_copy(hist_hbm, hist_ref)

    carry = jnp.int32(0)
    for bv in range(NB // 16):
        tot = jnp.zeros((16,), jnp.int32)
        pre = jnp.zeros((16,), jnp.int32)
        for wi in range(NW):
            h = hist_ref[pl.ds(wi * NB + bv * 16, 16)]
            tot = tot + h
            pre = pre + jnp.where(_splat(wi) < wv, h, jnp.zeros((16,), jnp.int32))
        tota = jnp.bitwise_and(tot + 7, _splat(-8))
        cs = plsc.cumsum(tota)
        cur_ref[pl.ds(bv * 16, 16)] = cs - tota + _splat(carry) + pre
        carry = carry + jnp.sum(tota)

    base = w * QV
    nrows = WIN * 16 // IDXB

    def _emit_vreg(srcv, dstv, lenv, mask):
        bucket = srcv >> 4
        cellv = ((srcv & 15) << 12) | dstv
        bidx = jnp.minimum(jnp.maximum(lenv, 1), MAXPD) - 1
        packed = (cellv << 5) | bidx
        if mask is None:
            cnt, _ = plsc.scan_count(bucket)
            cur = plsc.load_gather(cur_ref, [bucket])
            dest = cur + cnt - 1
            plsc.store_scatter(cur_ref, [bucket], cur + cnt)
        else:
            cnt, _ = plsc.scan_count(bucket, mask=mask)
            cur = plsc.load_gather(cur_ref, [bucket])
            dest = cur + cnt - 1
            plsc.store_scatter(cur_ref, [bucket], cur + cnt, mask=mask)
            pad = _splat(NPAD - 16) + lax.iota(jnp.int32, 16)
            dest = jnp.where(mask, dest, pad)
        return packed, dest

    def _do_window(j, slot):
        off = (base + j * WIN) * 16
        h1 = pltpu.async_copy(src_hbm.at[pl.ds(off, WIN * 16)], swin, sem)
        h2 = pltpu.async_copy(dst_hbm.at[pl.ds(off, WIN * 16)], dwin, sem)
        h3 = pltpu.async_copy(len_hbm.at[pl.ds(off, WIN * 16)], lwin, sem)
        h1.wait(); h2.wait(); h3.wait()
        for v in range(WIN):
            packed, dest = _emit_vreg(
                swin[pl.ds(v * 16, 16)], dwin[pl.ds(v * 16, 16)],
                lwin[pl.ds(v * 16, 16)], None)
            r, q = (v * 16) // IDXB, (v * 16) % IDXB
            pstage[slot, r, pl.ds(q, 16)] = packed
            ostage[slot, r, pl.ds(q, 16)] = dest
        for r in range(nrows):
            pltpu.async_copy(pstage.at[slot, r],
                             pair_out.at[ostage.at[slot, r]], osem)

    def _drain(slot):
        for r in range(nrows):
            pltpu.make_async_copy(pstage.at[slot, r],
                                  pair_out.at[ostage.at[slot, r]], osem).wait()

    def pairwin(i, carry):
        @pl.when(i > 0)
        def _():
            _drain(0)
        _do_window(2 * i, 0)

        @pl.when(i > 0)
        def _():
            _drain(1)
        _do_window(2 * i + 1, 1)
        return carry

    lax.fori_loop(0, NFULL // 2, pairwin, 0)
    _drain(0)
    _drain(1)

    tb = jnp.minimum(base + NFULL * WIN, VREGS - TAIL)
    vlo = base + NFULL * WIN - tb
    pltpu.sync_copy(src_hbm.at[pl.ds(tb * 16, TAIL * 16)],
                    swin.at[pl.ds(0, TAIL * 16)])
    pltpu.sync_copy(dst_hbm.at[pl.ds(tb * 16, TAIL * 16)],
                    dwin.at[pl.ds(0, TAIL * 16)])
    pltpu.sync_copy(len_hbm.at[pl.ds(tb * 16, TAIL * 16)],
                    lwin.at[pl.ds(0, TAIL * 16)])
    for v in range(TAIL):
        m = _splat(v) >= _splat(vlo)
        packed, dest = _emit_vreg(
            swin[pl.ds(v * 16, 16)], dwin[pl.ds(v * 16, 16)],
            lwin[pl.ds(v * 16, 16)], m)
        r, q = (v * 16) // IDXB, (v * 16) % IDXB
        pstage[0, r, pl.ds(q, 16)] = packed
        ostage[0, r, pl.ds(q, 16)] = dest
    for v in range(TAIL, ((TAIL * 16 + IDXB - 1) // IDXB) * IDXB // 16):
        r, q = (v * 16) // IDXB, (v * 16) % IDXB
        ostage[0, r, pl.ds(q, 16)] = _splat(NPAD - 16) + lax.iota(jnp.int32, 16)
    handles = []
    for r in range(-(-TAIL * 16 // IDXB)):
        handles.append(pltpu.async_copy(
            pstage.at[0, r], pair_out.at[ostage.at[0, r]], osem))
    for h in handles:
        h.wait()


@functools.partial(
    pl.kernel,
    out_type=jax.ShapeDtypeStruct((N * N,), jnp.float32),
    mesh=_mesh,
    compiler_params=_params,
    scratch_types=[
        pltpu.VMEM((CHUNK,), jnp.float32),
        pltpu.VMEM((NW * NB,), jnp.int32),
        pltpu.VMEM((NB,), jnp.int32),
        pltpu.VMEM((NB,), jnp.int32),
        pltpu.VMEM((W3,), jnp.int32),
        pltpu.VMEM((32,), jnp.float32),
        pltpu.SemaphoreType.DMA,
    ],
)
def _k3(pair_hbm, hist_hbm, b_hbm, out_hbm,
        chunk, hist_ref, off_ref, tot_ref, pwin, b_ref, sem):
    w = _wid()
    pltpu.sync_copy(hist_hbm, hist_ref)
    pltpu.sync_copy(b_hbm, b_ref.at[pl.ds(0, MAXPD)])

    carry = jnp.int32(0)
    for bv in range(NB // 16):
        tot = jnp.zeros((16,), jnp.int32)
        for wi in range(NW):
            tot = tot + hist_ref[pl.ds(wi * NB + bv * 16, 16)]
        tota = jnp.bitwise_and(tot + 7, _splat(-8))
        cs = plsc.cumsum(tota)
        off_ref[pl.ds(bv * 16, 16)] = cs - tota + _splat(carry)
        tot_ref[pl.ds(bv * 16, 16)] = tot
        carry = carry + jnp.sum(tota)

    def zbody(i, carry):
        for k in range(16):
            chunk[pl.ds(i * 256 + k * 16, 16)] = jnp.zeros((16,), jnp.float32)
        return carry

    lax.fori_loop(0, CHUNK // 256, zbody, 0)

    lane = lax.iota(jnp.int32, 16)
    zerof = jnp.zeros((16,), jnp.float32)

    def round_body(r, rcarry):
        b = w * (NB // NW) + r
        start = pl.multiple_of(_sget(off_ref, b), 8)
        tot = _sget(tot_ref, b)
        nwin = (tot + W3 - 1) // W3

        def swin_body(j, carry):
            rem = tot - j * W3
            pltpu.sync_copy(pair_hbm.at[pl.ds(start + j * W3, W3)], pwin)
            for v in range(W3 // 16):
                m = (_splat(v * 16) + lane) < _splat(rem)
                p = pwin[pl.ds(v * 16, 16)]
                val = plsc.load_gather(b_ref, [p & 31])
                plsc.store_scatter(chunk, [p >> 5], val, mask=m)
            return carry

        lax.fori_loop(0, nwin, swin_body, 0)

        pltpu.sync_copy(chunk,
                        out_hbm.at[pl.ds(pl.multiple_of(b * CHUNK, 8), CHUNK)])

        def zwin_body(j, carry):
            rem = tot - j * W3
            pltpu.sync_copy(pair_hbm.at[pl.ds(start + j * W3, W3)], pwin)
            for v in range(W3 // 16):
                m = (_splat(v * 16) + lane) < _splat(rem)
                plsc.store_scatter(chunk, [pwin[pl.ds(v * 16, 16)] >> 5],
                                   zerof, mask=m)
            return carry

        lax.fori_loop(0, nwin, zwin_body, 0)
        return rcarry

    lax.fori_loop(0, NB // NW, round_body, 0)


def kernel(x, path_src, path_dst, path_len, b):
    hist = _k1(path_src)
    pair = _k2(path_src, path_dst, path_len, hist)
    flat = _k3(pair, hist, b)
    return flat.reshape(N, N).astype(x.dtype)


# speedup vs baseline: 6.2248x; 2.8942x over previous
import functools

import jax
import jax.numpy as jnp
from jax import lax
from jax.experimental import pallas as pl
from jax.experimental.pallas import tpu as pltpu, tpu_sc as plsc

N = 4096
NP = 1_000_000
MAXPD = 20
NW = 32
NG = 32
GROWS = N // NG
RROWS = 16
CHUNK = RROWS * N
NRND = GROWS // RROWS
VREGS = NP // 16
QV = -(-VREGS // NW)
WIN = 64
NFULL = QV // WIN
TAIL = QV - NFULL * WIN
WELEM = QV * 16
REGION = WELEM + NG * 8
NPAD = NW * REGION + 2048
W3 = 2048
SCAP = 57344

_mesh = plsc.VectorSubcoreMesh(core_axis_name="c", subcore_axis_name="s")
_params = pltpu.CompilerParams(needs_layout_passes=False)


def _wid():
    return lax.axis_index("c") * 16 + lax.axis_index("s")


def _splat(x):
    return jnp.full((16,), x, jnp.int32)


def _vext0(vec):
    lane = lax.iota(jnp.int32, 16)
    return jnp.sum(jnp.where(lane == 0, vec, jnp.zeros((16,), vec.dtype)))


def _sget(ref, idx_scalar):
    return _vext0(plsc.load_gather(ref, [_splat(idx_scalar)]))


@functools.partial(
    pl.kernel,
    out_type=jax.ShapeDtypeStruct((NW * NG,), jnp.int32),
    mesh=_mesh,
    compiler_params=_params,
    scratch_types=[
        pltpu.VMEM((WIN * 16,), jnp.int32),
        pltpu.VMEM((NG,), jnp.int32),
    ],
)
def _k1(src_hbm, hist_out, win_ref, hist_ref):
    w = _wid()
    base = w * QV
    ones = jnp.ones((16,), jnp.int32)
    for i in range(NG // 16):
        hist_ref[pl.ds(i * 16, 16)] = jnp.zeros((16,), jnp.int32)

    def fullwin(j, carry):
        pltpu.sync_copy(src_hbm.at[pl.ds((base + j * WIN) * 16, WIN * 16)],
                        win_ref)
        for v in range(WIN):
            srcv = win_ref[pl.ds(v * 16, 16)]
            plsc.addupdate_scatter(hist_ref, [srcv >> 7], ones)
        return carry

    lax.fori_loop(0, NFULL, fullwin, 0)

    tb = jnp.minimum(base + NFULL * WIN, VREGS - TAIL)
    vlo = base + NFULL * WIN - tb
    pltpu.sync_copy(src_hbm.at[pl.ds(tb * 16, TAIL * 16)],
                    win_ref.at[pl.ds(0, TAIL * 16)])
    for v in range(TAIL):
        m = _splat(v) >= _splat(vlo)
        srcv = win_ref[pl.ds(v * 16, 16)]
        plsc.addupdate_scatter(hist_ref, [srcv >> 7], ones, mask=m)

    pltpu.sync_copy(hist_ref, hist_out.at[pl.ds(w * NG, NG)])


@functools.partial(
    pl.kernel,
    out_type=jax.ShapeDtypeStruct((NPAD,), jnp.int32),
    mesh=_mesh,
    compiler_params=_params,
    scratch_types=[
        pltpu.VMEM((NG,), jnp.int32),
        pltpu.VMEM((NG,), jnp.int32),
        pltpu.VMEM((WIN * 16,), jnp.int32),
        pltpu.VMEM((WIN * 16,), jnp.int32),
        pltpu.VMEM((WIN * 16,), jnp.int32),
        pltpu.VMEM((REGION,), jnp.int32),
        pltpu.SemaphoreType.DMA,
    ],
)
def _k2(src_hbm, dst_hbm, len_hbm, hist_hbm, pair_out,
        hist_ref, cur_ref, swin, dwin, lwin, stage, sem):
    w = _wid()
    pltpu.sync_copy(hist_hbm.at[pl.ds(w * NG, NG)], hist_ref)

    for gv in range(NG // 16):
        h = hist_ref[pl.ds(gv * 16, 16)]
        ha = jnp.bitwise_and(h + 7, _splat(-8))
        cs = plsc.cumsum(ha)
        if gv == 0:
            cur_ref[pl.ds(0, 16)] = cs - ha
            carry = jnp.sum(ha)
        else:
            cur_ref[pl.ds(16, 16)] = cs - ha + _splat(carry)

    base = w * QV

    def _emit_vreg(srcv, dstv, lenv, mask):
        group = srcv >> 7
        cellv = ((srcv & 127) << 12) | dstv
        bidx = jnp.minimum(jnp.maximum(lenv, 1), MAXPD) - 1
        packed = (cellv << 5) | bidx
        cnt, _ = plsc.scan_count(group, mask=mask)
        cur = plsc.load_gather(cur_ref, [group])
        dest = cur + cnt - 1
        plsc.store_scatter(cur_ref, [group], cur + cnt, mask=mask)
        plsc.store_scatter(stage, [dest], packed, mask=mask)

    def window(j, carry):
        off = (base + j * WIN) * 16
        h1 = pltpu.async_copy(src_hbm.at[pl.ds(off, WIN * 16)], swin, sem)
        h2 = pltpu.async_copy(dst_hbm.at[pl.ds(off, WIN * 16)], dwin, sem)
        h3 = pltpu.async_copy(len_hbm.at[pl.ds(off, WIN * 16)], lwin, sem)
        h1.wait(); h2.wait(); h3.wait()
        for v in range(WIN):
            _emit_vreg(swin[pl.ds(v * 16, 16)], dwin[pl.ds(v * 16, 16)],
                       lwin[pl.ds(v * 16, 16)], None)
        return carry

    lax.fori_loop(0, NFULL, window, 0)

    tb = jnp.minimum(base + NFULL * WIN, VREGS - TAIL)
    vlo = base + NFULL * WIN - tb
    pltpu.sync_copy(src_hbm.at[pl.ds(tb * 16, TAIL * 16)],
                    swin.at[pl.ds(0, TAIL * 16)])
    pltpu.sync_copy(dst_hbm.at[pl.ds(tb * 16, TAIL * 16)],
                    dwin.at[pl.ds(0, TAIL * 16)])
    pltpu.sync_copy(len_hbm.at[pl.ds(tb * 16, TAIL * 16)],
                    lwin.at[pl.ds(0, TAIL * 16)])
    for v in range(TAIL):
        m = _splat(v) >= _splat(vlo)
        _emit_vreg(swin[pl.ds(v * 16, 16)], dwin[pl.ds(v * 16, 16)],
                   lwin[pl.ds(v * 16, 16)], m)

    pltpu.sync_copy(stage, pair_out.at[pl.ds(w * REGION, REGION)])


@functools.partial(
    pl.kernel,
    out_type=jax.ShapeDtypeStruct((N * N,), jnp.float32),
    mesh=_mesh,
    compiler_params=_params,
    scratch_types=[
        pltpu.VMEM((CHUNK,), jnp.float32),
        pltpu.VMEM((NW * NG,), jnp.int32),
        pltpu.VMEM((NW,), jnp.int32),
        pltpu.VMEM((NW,), jnp.int32),
        pltpu.VMEM((SCAP,), jnp.int32),
        pltpu.VMEM((W3,), jnp.int32),
        pltpu.VMEM((32,), jnp.float32),
        pltpu.SemaphoreType.DMA,
    ],
)
def _k3(pair_hbm, hist_hbm, b_hbm, out_hbm,
        chunk, hist_ref, rst_ref, rlen_ref, stg, pwin, b_ref, sem):
    w = _wid()
    pltpu.sync_copy(hist_hbm, hist_ref)
    pltpu.sync_copy(b_hbm, b_ref.at[pl.ds(0, MAXPD)])
    lane = lax.iota(jnp.int32, 16)
    zerof = jnp.zeros((16,), jnp.float32)

    total = jnp.int32(0)
    for wi in range(NW):
        h = hist_ref[pl.ds(wi * NG, 16)]
        h2 = hist_ref[pl.ds(wi * NG + 16, 16)]
        ha = jnp.bitwise_and(h + 7, _splat(-8))
        ha2 = jnp.bitwise_and(h2 + 7, _splat(-8))
        cs = plsc.cumsum(ha)
        cs2 = plsc.cumsum(ha2) + _splat(jnp.sum(ha))
        loff = jnp.where(_splat(w) < 16,
                         jnp.sum(jnp.where(lane == (w & 15), cs - ha, 0)),
                         jnp.sum(jnp.where(lane == (w & 15), cs2 - ha2, 0)))
        mylen = jnp.sum(jnp.where(lane == (w & 15),
                                  jnp.where(_splat(w) < 16, h, h2), 0))
        rst_ref[pl.ds((wi // 16) * 16, 16)] = jnp.where(
            lane == (wi & 15), _splat(wi * REGION + loff),
            rst_ref[pl.ds((wi // 16) * 16, 16)])
        rlen_ref[pl.ds((wi // 16) * 16, 16)] = jnp.where(
            lane == (wi & 15), _splat(mylen),
            rlen_ref[pl.ds((wi // 16) * 16, 16)])
        total = total + mylen

    fits = total <= SCAP

    @pl.when(fits)
    def _():
        def stage_run(wi, woff):
            st = pl.multiple_of(_sget(rst_ref, wi), 8)
            ln = _sget(rlen_ref, wi)
            nw_ = (ln + W3 - 1) // W3

            def cpw(j, woff):
                pltpu.sync_copy(pair_hbm.at[pl.ds(st + j * W3, W3)], pwin)
                rem = ln - j * W3
                for v in range(W3 // 16):
                    m = (_splat(v * 16) + lane) < _splat(rem)
                    dest = _splat(woff + v * 16) + lane
                    plsc.store_scatter(stg, [dest],
                                       pwin[pl.ds(v * 16, 16)], mask=m)
                return woff + jnp.minimum(rem, W3)

            return lax.fori_loop(0, nw_, cpw, woff)

        lax.fori_loop(0, NW, stage_run, jnp.int32(0))

    def zbody(i, carry):
        for k in range(16):
            chunk[pl.ds(i * 256 + k * 16, 16)] = jnp.zeros((16,), jnp.float32)
        return carry

    lax.fori_loop(0, CHUNK // 256, zbody, 0)

    def round_body(r, rcarry):
        lo = r * CHUNK

        def scat_vreg(p, m):
            cell = (p >> 5) - _splat(lo)
            m2 = m & (cell >= 0) & (cell < CHUNK)
            val = plsc.load_gather(b_ref, [p & 31])
            plsc.store_scatter(chunk, [cell], val, mask=m2)

        def zero_vreg(p, m):
            cell = (p >> 5) - _splat(lo)
            m2 = m & (cell >= 0) & (cell < CHUNK)
            plsc.store_scatter(chunk, [cell], zerof, mask=m2)

        def pass_over(do_vreg):
            @pl.when(fits)
            def _():
                nv = (total + 15) // 16

                def body(i, carry):
                    m = (_splat(0) + lane) < _splat(total - i * 16)
                    do_vreg(plsc.load_gather(
                        stg, [_splat(i * 16) + lane]), m)
                    return carry

                lax.fori_loop(0, nv, body, 0)

            @pl.when(jnp.logical_not(fits))
            def _():
                def per_run(wi, carry):
                    st = pl.multiple_of(_sget(rst_ref, wi), 8)
                    ln = _sget(rlen_ref, wi)
                    nw_ = (ln + W3 - 1) // W3

                    def body(j, carry):
                        pltpu.sync_copy(
                            pair_hbm.at[pl.ds(st + j * W3, W3)], pwin)
                        rem = ln - j * W3
                        for v in range(W3 // 16):
                            m = (_splat(v * 16) + lane) < _splat(rem)
                            do_vreg(pwin[pl.ds(v * 16, 16)], m)
                        return carry

                    return lax.fori_loop(0, nw_, body, carry)

                lax.fori_loop(0, NW, per_run, 0)

        pass_over(scat_vreg)
        pltpu.sync_copy(
            chunk,
            out_hbm.at[pl.ds(pl.multiple_of(w * GROWS * N + r * CHUNK, 8),
                             CHUNK)])
        pass_over(zero_vreg)
        return rcarry

    lax.fori_loop(0, NRND, round_body, 0)


def kernel(x, path_src, path_dst, path_len, b):
    hist = _k1(path_src)
    pair = _k2(path_src, path_dst, path_len, hist)
    flat = _k3(pair, hist, b)
    return flat.reshape(N, N).astype(x.dtype)
